# baseline (speedup 1.0000x reference)
import functools

import jax
import jax.numpy as jnp
from jax import lax
from jax.experimental import pallas as pl
from jax.experimental.pallas import tpu as pltpu
from jax.experimental.pallas import tpu_sc as plsc

D = 128
CHUNK = 64
NBUF = 8


@functools.lru_cache(maxsize=None)
def _make_gather(total_rows: int):
    info = plsc.get_sparse_core_info()
    nw = info.num_cores * info.num_subcores
    assert total_rows % (nw * CHUNK * NBUF) == 0
    rows_per_w = total_rows // nw
    ngroups = rows_per_w // CHUNK
    nsteps = ngroups // NBUF
    mesh = plsc.VectorSubcoreMesh(core_axis_name="c", subcore_axis_name="s")

    @functools.partial(
        pl.kernel,
        mesh=mesh,
        out_type=jax.ShapeDtypeStruct((total_rows, D), jnp.float32),
        scratch_types=[
            pltpu.VMEM((rows_per_w,), jnp.int32),
            pltpu.VMEM((NBUF, CHUNK, D), jnp.float32),
        ]
        + [pltpu.SemaphoreType.DMA] * NBUF
        + [pltpu.SemaphoreType.DMA] * NBUF,
    )
    def gather_kernel(idx_hbm, table_hbm, out_hbm, idx_v, rows_v, *sems):
        gsem = sems[:NBUF]
        ssem = sems[NBUF:]
        wid = lax.axis_index("s") * info.num_cores + lax.axis_index("c")
        base = pl.multiple_of(wid * rows_per_w, 8)
        pltpu.sync_copy(idx_hbm.at[pl.ds(base, rows_per_w)], idx_v)

        def gather_start(g, b):
            off = pl.multiple_of(g * CHUNK, 8)
            pltpu.async_copy(
                table_hbm.at[idx_v.at[pl.ds(off, CHUNK)]], rows_v.at[b], gsem[b]
            )

        def gather_wait(g, b):
            off = pl.multiple_of(g * CHUNK, 8)
            pltpu.make_async_copy(
                table_hbm.at[idx_v.at[pl.ds(off, CHUNK)]], rows_v.at[b], gsem[b]
            ).wait()

        def store_start(g, b):
            off = pl.multiple_of(g * CHUNK, 8)
            pltpu.async_copy(
                rows_v.at[b], out_hbm.at[pl.ds(base + off, CHUNK)], ssem[b]
            )

        def store_wait(g, b):
            off = pl.multiple_of(g * CHUNK, 8)
            pltpu.make_async_copy(
                rows_v.at[b], out_hbm.at[pl.ds(base + off, CHUNK)], ssem[b]
            ).wait()

        for b in range(NBUF):
            gather_start(b, b)
        for b in range(NBUF):
            gather_wait(b, b)
            store_start(b, b)

        def body(p, carry):
            g0 = p * NBUF
            for b in range(NBUF):
                store_wait(g0 + b - NBUF, b)
                gather_start(g0 + b, b)
            for b in range(NBUF):
                gather_wait(g0 + b, b)
                store_start(g0 + b, b)
            return carry

        lax.fori_loop(1, nsteps, body, 0)

        gl = (nsteps - 1) * NBUF
        for b in range(NBUF):
            store_wait(gl + b, b)

    return gather_kernel


def kernel(a, b, embedding_table):
    batch, hist = a.shape
    idx = jnp.stack([a.T.astype(jnp.int32), b.T.astype(jnp.int32)])
    flat = _make_gather(2 * batch * hist)(idx.reshape(-1), embedding_table)
    return flat.reshape(2, hist, batch, D).transpose(0, 2, 1, 3)


# speedup vs baseline: 1.0005x; 1.0005x over previous
import functools

import jax
import jax.numpy as jnp
from jax import lax
from jax.experimental import pallas as pl
from jax.experimental.pallas import tpu as pltpu
from jax.experimental.pallas import tpu_sc as plsc

D = 128
CHUNK = 64
NBUF = 8


@functools.lru_cache(maxsize=None)
def _make_gather(total_rows: int):
    info = plsc.get_sparse_core_info()
    nw = info.num_cores * info.num_subcores
    assert total_rows % (nw * CHUNK * NBUF) == 0
    rows_per_w = total_rows // nw
    ngroups = rows_per_w // CHUNK
    nsteps = ngroups // NBUF
    mesh = plsc.VectorSubcoreMesh(core_axis_name="c", subcore_axis_name="s")

    @functools.partial(
        pl.kernel,
        mesh=mesh,
        out_type=jax.ShapeDtypeStruct((total_rows, D), jnp.float32),
        scratch_types=[
            pltpu.VMEM((rows_per_w,), jnp.int32),
            pltpu.VMEM((NBUF, CHUNK, D), jnp.float32),
        ]
        + [pltpu.SemaphoreType.DMA] * NBUF
        + [pltpu.SemaphoreType.DMA] * NBUF,
    )
    def gather_kernel(idx_hbm, table_hbm, out_hbm, idx_v, rows_v, *sems):
        gsem = sems[:NBUF]
        ssem = sems[NBUF:]
        wid = lax.axis_index("s") * info.num_cores + lax.axis_index("c")
        base = pl.multiple_of(wid * rows_per_w, 8)
        pltpu.sync_copy(idx_hbm.at[pl.ds(base, rows_per_w)], idx_v)

        rot = lax.rem(wid, 2) * (ngroups // 2)

        def eff(g):
            return lax.rem(g + rot, ngroups)

        def gather_start(g, b):
            off = pl.multiple_of(eff(g) * CHUNK, 8)
            pltpu.async_copy(
                table_hbm.at[idx_v.at[pl.ds(off, CHUNK)]], rows_v.at[b], gsem[b]
            )

        def gather_wait(g, b):
            off = pl.multiple_of(eff(g) * CHUNK, 8)
            pltpu.make_async_copy(
                table_hbm.at[idx_v.at[pl.ds(off, CHUNK)]], rows_v.at[b], gsem[b]
            ).wait()

        def store_start(g, b):
            off = pl.multiple_of(eff(g) * CHUNK, 8)
            pltpu.async_copy(
                rows_v.at[b], out_hbm.at[pl.ds(base + off, CHUNK)], ssem[b]
            )

        def store_wait(g, b):
            off = pl.multiple_of(eff(g) * CHUNK, 8)
            pltpu.make_async_copy(
                rows_v.at[b], out_hbm.at[pl.ds(base + off, CHUNK)], ssem[b]
            ).wait()

        for b in range(NBUF):
            gather_start(b, b)
        for b in range(NBUF):
            gather_wait(b, b)
            store_start(b, b)

        def body(p, carry):
            g0 = p * NBUF
            for b in range(NBUF):
                store_wait(g0 + b - NBUF, b)
                gather_start(g0 + b, b)
            for b in range(NBUF):
                gather_wait(g0 + b, b)
                store_start(g0 + b, b)
            return carry

        lax.fori_loop(1, nsteps, body, 0)

        gl = (nsteps - 1) * NBUF
        for b in range(NBUF):
            store_wait(gl + b, b)

    return gather_kernel


def kernel(a, b, embedding_table):
    batch, hist = a.shape
    idx = jnp.stack([a.T.astype(jnp.int32), b.T.astype(jnp.int32)])
    flat = _make_gather(2 * batch * hist)(idx.reshape(-1), embedding_table)
    return flat.reshape(2, hist, batch, D).transpose(0, 2, 1, 3)
